# Initial kernel scaffold; baseline (speedup 1.0000x reference)
import jax
import jax.numpy as jnp
from jax.experimental import pallas as pl


def kernel(cat_features, num_features, cat_linear_weight, num_linear, bias):
    raise NotImplementedError("write your pallas kernel here")


# speedup vs baseline: 1.3592x; 1.3592x over previous
import functools

import jax
import jax.numpy as jnp
from jax import lax
from jax.experimental import pallas as pl
from jax.experimental.pallas import tpu as pltpu
from jax.experimental.pallas import tpu_sc as plsc

B = 16384
F = 26
K = 13
V = 1000000

_info = plsc.get_sparse_core_info()
_NC, _NS = _info.num_cores, _info.num_subcores
_NW = _NC * _NS
_BW = B // _NW
_NIDX = F * _BW
_NROW = _NIDX // 128
_NCHUNK = _BW // 16

_mesh = plsc.VectorSubcoreMesh(core_axis_name="c", subcore_axis_name="s")


@functools.partial(
    pl.kernel,
    out_type=jax.ShapeDtypeStruct((B,), jnp.float32),
    mesh=_mesh,
    scratch_types=[
        pltpu.VMEM((_NROW, 128), jnp.int32),
        pltpu.VMEM((_NROW, 128), jnp.float32),
        pltpu.VMEM((K, _BW), jnp.float32),
        pltpu.VMEM((K, 16), jnp.float32),
        pltpu.VMEM((16,), jnp.float32),
        pltpu.VMEM((_BW,), jnp.float32),
        pltpu.SemaphoreType.DMA,
    ],
)
def _sc_forward(idx_hbm, num_hbm, table_hbm, wb_hbm, bias_hbm, out_hbm,
                idx_v, vals_v, num_v, wb_v, bias_v, acc_v, sem):
    wid = lax.axis_index("s") * _NC + lax.axis_index("c")

    pltpu.sync_copy(idx_hbm.at[wid], idx_v)
    pltpu.sync_copy(num_hbm.at[wid], num_v)
    pltpu.sync_copy(wb_hbm, wb_v)
    pltpu.sync_copy(bias_hbm, bias_v)

    def fire(j, carry):
        pltpu.make_async_copy(
            table_hbm.at[idx_v.at[j]], vals_v.at[j], sem).start()
        return carry

    lax.fori_loop(0, _NROW, fire, 0)

    def drain(j, carry):
        pltpu.make_async_copy(
            table_hbm.at[idx_v.at[j]], vals_v.at[j], sem).wait()
        return carry

    lax.fori_loop(0, _NROW, drain, 0)

    def chunk(c, carry):
        r = c // 8
        col = (c % 8) * 16
        acc = bias_v[:]
        for f in range(F):
            acc = acc + vals_v[4 * f + r, pl.ds(col, 16)]
        for k in range(K):
            acc = acc + num_v[k, pl.ds(c * 16, 16)] * wb_v[k, :]
        acc_v[pl.ds(c * 16, 16)] = acc
        return carry

    lax.fori_loop(0, _NCHUNK, chunk, 0)

    pltpu.sync_copy(acc_v, out_hbm.at[pl.ds(wid * _BW, _BW)])


def kernel(cat_features, num_features, cat_linear_weight, num_linear, bias):
    cat = cat_features.astype(jnp.int32)
    idx_r = (cat.T.reshape(F, _NW, _BW).transpose(1, 0, 2)
             .reshape(_NW, _NROW, 128))
    num_r = (num_features.astype(jnp.float32).T
             .reshape(K, _NW, _BW).transpose(1, 0, 2))
    table = cat_linear_weight.astype(jnp.float32).reshape(V)
    wb = jnp.broadcast_to(num_linear.astype(jnp.float32).reshape(K, 1),
                          (K, 16))
    bias16 = jnp.broadcast_to(bias.astype(jnp.float32), (16,))
    out = _sc_forward(idx_r, num_r, table, wb, bias16)
    return out.reshape(B, 1)


# speedup vs baseline: 2.3778x; 1.7494x over previous
import functools

import jax
import jax.numpy as jnp
from jax import lax
from jax.experimental import pallas as pl
from jax.experimental.pallas import tpu as pltpu
from jax.experimental.pallas import tpu_sc as plsc

B = 16384
F = 26
K = 13
V = 1000000

_info = plsc.get_sparse_core_info()
_NC, _NS = _info.num_cores, _info.num_subcores
_NW = _NC * _NS
_BW = B // _NW
_NIDX = F * _BW
_NCHUNK = _BW // 16

_mesh = plsc.VectorSubcoreMesh(core_axis_name="c", subcore_axis_name="s")


@functools.partial(
    pl.kernel,
    out_type=jax.ShapeDtypeStruct((B,), jnp.float32),
    mesh=_mesh,
    scratch_types=[
        pltpu.VMEM((_NIDX,), jnp.int32),
        pltpu.VMEM((_NIDX,), jnp.float32),
        pltpu.VMEM((_BW,), jnp.float32),
        pltpu.SemaphoreType.DMA,
        pltpu.SemaphoreType.DMA,
    ],
)
def _sc_gather_sum(idx_hbm, table_hbm, out_hbm,
                   idx_v, vals_v, acc_v, sem, sem2):
    wid = lax.axis_index("s") * _NC + lax.axis_index("c")

    pltpu.sync_copy(idx_hbm.at[wid], idx_v)

    table_flat = table_hbm.at[0]
    _HALF = _NIDX // 2
    _FH = F // 2
    lo = pl.ds(0, _HALF)
    hi = pl.ds(_HALF, _HALF)
    pltpu.make_async_copy(table_flat.at[idx_v.at[lo]], vals_v.at[lo],
                          sem).start()
    pltpu.make_async_copy(table_flat.at[idx_v.at[hi]], vals_v.at[hi],
                          sem2).start()

    pltpu.make_async_copy(table_flat.at[idx_v.at[lo]], vals_v.at[lo],
                          sem).wait()

    def chunk_lo(c, carry):
        acc = vals_v[pl.ds(c * 16, 16)]
        for f in range(1, _FH):
            acc = acc + vals_v[pl.ds(f * _BW + c * 16, 16)]
        acc_v[pl.ds(c * 16, 16)] = acc
        return carry

    lax.fori_loop(0, _NCHUNK, chunk_lo, 0)

    pltpu.make_async_copy(table_flat.at[idx_v.at[hi]], vals_v.at[hi],
                          sem2).wait()

    def chunk_hi(c, carry):
        acc = acc_v[pl.ds(c * 16, 16)]
        for f in range(_FH, F):
            acc = acc + vals_v[pl.ds(f * _BW + c * 16, 16)]
        acc_v[pl.ds(c * 16, 16)] = acc
        return carry

    lax.fori_loop(0, _NCHUNK, chunk_hi, 0)

    pltpu.sync_copy(acc_v, out_hbm.at[pl.ds(wid * _BW, _BW)])


def _tc_combine_body(cat_ref, num_ref, w_ref, b_ref, out_ref):
    out_ref[...] = (cat_ref[...]
                    + jnp.sum(num_ref[...] * w_ref[...], axis=1)
                    + b_ref[0])


_BBLK = 2048


def _tc_combine(cat_sums, num_features, num_linear, bias):
    return pl.pallas_call(
        _tc_combine_body,
        out_shape=jax.ShapeDtypeStruct((B,), jnp.float32),
        grid=(B // _BBLK,),
        in_specs=[
            pl.BlockSpec((_BBLK,), lambda i: (i,)),
            pl.BlockSpec((_BBLK, K), lambda i: (i, 0)),
            pl.BlockSpec((1, K), lambda i: (0, 0)),
            pl.BlockSpec(memory_space=pltpu.SMEM),
        ],
        out_specs=pl.BlockSpec((_BBLK,), lambda i: (i,)),
    )(cat_sums, num_features, num_linear, bias)


def kernel(cat_features, num_features, cat_linear_weight, num_linear, bias):
    cat = cat_features.astype(jnp.int32)
    idx_r = (cat.reshape(_NW, _BW, F).transpose(0, 2, 1)
             .reshape(_NW, _NIDX))
    table2 = cat_linear_weight.astype(jnp.float32).reshape(1, V)
    cat_sums = _sc_gather_sum(idx_r, table2)
    out = _tc_combine(cat_sums, num_features.astype(jnp.float32),
                      num_linear.astype(jnp.float32),
                      bias.astype(jnp.float32))
    return out.reshape(B, 1)


# speedup vs baseline: 2.7735x; 1.1664x over previous
import functools

import jax
import jax.numpy as jnp
from jax import lax
from jax.experimental import pallas as pl
from jax.experimental.pallas import tpu as pltpu
from jax.experimental.pallas import tpu_sc as plsc

B = 16384
F = 26
K = 13
V = 1000000

_info = plsc.get_sparse_core_info()
_NC, _NS = _info.num_cores, _info.num_subcores
_NW = _NC * _NS
_BW = B // _NW
_NIDX = F * _BW
_NCHUNK = _BW // 16

_mesh = plsc.VectorSubcoreMesh(core_axis_name="c", subcore_axis_name="s")


@functools.partial(
    pl.kernel,
    out_type=jax.ShapeDtypeStruct((B,), jnp.float32),
    mesh=_mesh,
    scratch_types=[
        pltpu.VMEM((_NIDX,), jnp.int32),
        pltpu.VMEM((_NIDX,), jnp.float32),
        pltpu.VMEM((K, _BW), jnp.float32),
        pltpu.VMEM((K, 16), jnp.float32),
        pltpu.VMEM((16,), jnp.float32),
        pltpu.VMEM((_BW,), jnp.float32),
        pltpu.SemaphoreType.DMA,
        pltpu.SemaphoreType.DMA,
    ],
)
def _sc_forward(idx_hbm, num_hbm, table_hbm, wb_hbm, bias_hbm, out_hbm,
                idx_v, vals_v, num_v, wb_v, bias_v, acc_v, sem, sem2):
    wid = lax.axis_index("s") * _NC + lax.axis_index("c")

    pltpu.sync_copy(idx_hbm.at[wid], idx_v)

    table_flat = table_hbm.at[0]
    _HALF = _NIDX // 2
    _FH = F // 2
    lo = pl.ds(0, _HALF)
    hi = pl.ds(_HALF, _HALF)
    pltpu.make_async_copy(table_flat.at[idx_v.at[lo]], vals_v.at[lo],
                          sem).start()
    pltpu.make_async_copy(table_flat.at[idx_v.at[hi]], vals_v.at[hi],
                          sem2).start()

    pltpu.sync_copy(num_hbm.at[wid], num_v)
    pltpu.sync_copy(wb_hbm, wb_v)
    pltpu.sync_copy(bias_hbm, bias_v)

    pltpu.make_async_copy(table_flat.at[idx_v.at[lo]], vals_v.at[lo],
                          sem).wait()

    def chunk_lo(c, carry):
        acc = bias_v[:]
        for f in range(_FH):
            acc = acc + vals_v[pl.ds(f * _BW + c * 16, 16)]
        for k in range(K):
            acc = acc + num_v[k, pl.ds(c * 16, 16)] * wb_v[k, :]
        acc_v[pl.ds(c * 16, 16)] = acc
        return carry

    lax.fori_loop(0, _NCHUNK, chunk_lo, 0)

    pltpu.make_async_copy(table_flat.at[idx_v.at[hi]], vals_v.at[hi],
                          sem2).wait()

    def chunk_hi(c, carry):
        acc = acc_v[pl.ds(c * 16, 16)]
        for f in range(_FH, F):
            acc = acc + vals_v[pl.ds(f * _BW + c * 16, 16)]
        acc_v[pl.ds(c * 16, 16)] = acc
        return carry

    lax.fori_loop(0, _NCHUNK, chunk_hi, 0)

    pltpu.sync_copy(acc_v, out_hbm.at[pl.ds(wid * _BW, _BW)])


def kernel(cat_features, num_features, cat_linear_weight, num_linear, bias):
    cat = cat_features.astype(jnp.int32)
    idx_r = (cat.reshape(_NW, _BW, F).transpose(0, 2, 1)
             .reshape(_NW, _NIDX))
    num_r = (num_features.astype(jnp.float32)
             .reshape(_NW, _BW, K).transpose(0, 2, 1))
    table2 = cat_linear_weight.astype(jnp.float32).reshape(1, V)
    wb = jnp.broadcast_to(num_linear.astype(jnp.float32).reshape(K, 1),
                          (K, 16))
    bias16 = jnp.broadcast_to(bias.astype(jnp.float32), (16,))
    out = _sc_forward(idx_r, num_r, table2, wb, bias16)
    return out.reshape(B, 1)


# speedup vs baseline: 2.8757x; 1.0368x over previous
import functools

import jax
import jax.numpy as jnp
from jax import lax
from jax.experimental import pallas as pl
from jax.experimental.pallas import tpu as pltpu
from jax.experimental.pallas import tpu_sc as plsc

B = 16384
F = 26
K = 13
V = 1000000

_info = plsc.get_sparse_core_info()
_NC, _NS = _info.num_cores, _info.num_subcores
_NW = _NC * _NS
_BW = B // _NW
_NIDX = F * _BW
_NCHUNK = _BW // 16

_mesh = plsc.VectorSubcoreMesh(core_axis_name="c", subcore_axis_name="s")


@functools.partial(
    pl.kernel,
    out_type=jax.ShapeDtypeStruct((B,), jnp.float32),
    mesh=_mesh,
    scratch_types=[
        pltpu.VMEM((_NIDX,), jnp.int32),
        pltpu.VMEM((_NIDX,), jnp.float32),
        pltpu.VMEM((_BW,), jnp.float32),
        pltpu.SemaphoreType.DMA,
        pltpu.SemaphoreType.DMA,
    ],
)
def _sc_gather_sum(idx_hbm, table_hbm, out_hbm,
                   idx_v, vals_v, acc_v, sem, sem2):
    wid = lax.axis_index("s") * _NC + lax.axis_index("c")

    pltpu.sync_copy(idx_hbm.at[wid], idx_v)

    table_flat = table_hbm.at[0]
    _HALF = _NIDX // 2
    _FH = F // 2
    lo = pl.ds(0, _HALF)
    hi = pl.ds(_HALF, _HALF)
    pltpu.make_async_copy(table_flat.at[idx_v.at[lo]], vals_v.at[lo],
                          sem).start()
    pltpu.make_async_copy(table_flat.at[idx_v.at[hi]], vals_v.at[hi],
                          sem2).start()

    pltpu.make_async_copy(table_flat.at[idx_v.at[lo]], vals_v.at[lo],
                          sem).wait()

    def chunk_lo(c, carry):
        acc = vals_v[pl.ds(c * 16, 16)]
        for f in range(1, _FH):
            acc = acc + vals_v[pl.ds(f * _BW + c * 16, 16)]
        acc_v[pl.ds(c * 16, 16)] = acc
        return carry

    lax.fori_loop(0, _NCHUNK, chunk_lo, 0)

    pltpu.make_async_copy(table_flat.at[idx_v.at[hi]], vals_v.at[hi],
                          sem2).wait()

    def chunk_hi(c, carry):
        acc = acc_v[pl.ds(c * 16, 16)]
        for f in range(_FH, F):
            acc = acc + vals_v[pl.ds(f * _BW + c * 16, 16)]
        acc_v[pl.ds(c * 16, 16)] = acc
        return carry

    lax.fori_loop(0, _NCHUNK, chunk_hi, 0)

    pltpu.sync_copy(acc_v, out_hbm.at[pl.ds(wid * _BW, _BW)])


def _tc_combine_body(cat_ref, numt_ref, w_ref, b_ref, out_ref):
    acc = cat_ref[...] + b_ref[0]
    numt = numt_ref[...]
    for k in range(K):
        acc = acc + numt[k, :] * w_ref[0, k]
    out_ref[...] = acc


_BBLK = 4096


def _tc_combine(cat_sums, num_t, num_linear, bias):
    return pl.pallas_call(
        _tc_combine_body,
        out_shape=jax.ShapeDtypeStruct((B,), jnp.float32),
        grid=(B // _BBLK,),
        in_specs=[
            pl.BlockSpec((_BBLK,), lambda i: (i,)),
            pl.BlockSpec((K, _BBLK), lambda i: (0, i)),
            pl.BlockSpec(memory_space=pltpu.SMEM),
            pl.BlockSpec(memory_space=pltpu.SMEM),
        ],
        out_specs=pl.BlockSpec((_BBLK,), lambda i: (i,)),
    )(cat_sums, num_t, num_linear, bias)


def kernel(cat_features, num_features, cat_linear_weight, num_linear, bias):
    cat = cat_features.astype(jnp.int32)
    idx_r = (cat.reshape(_NW, _BW, F).transpose(0, 2, 1)
             .reshape(_NW, _NIDX))
    table2 = cat_linear_weight.astype(jnp.float32).reshape(1, V)
    cat_sums = _sc_gather_sum(idx_r, table2)
    num_t = num_features.astype(jnp.float32).T
    out = _tc_combine(cat_sums, num_t, num_linear.astype(jnp.float32),
                      bias.astype(jnp.float32))
    return out.reshape(B, 1)


# speedup vs baseline: 2.9445x; 1.0239x over previous
import functools

import jax
import jax.numpy as jnp
from jax import lax
from jax.experimental import pallas as pl
from jax.experimental.pallas import tpu as pltpu
from jax.experimental.pallas import tpu_sc as plsc

B = 16384
F = 26
K = 13
V = 1000000

_info = plsc.get_sparse_core_info()
_NC, _NS = _info.num_cores, _info.num_subcores
_NW = _NC * _NS
_BW = B // _NW
_NIDX = F * _BW
_NCHUNK = _BW // 16

_mesh = plsc.VectorSubcoreMesh(core_axis_name="c", subcore_axis_name="s")


@functools.partial(
    pl.kernel,
    out_type=jax.ShapeDtypeStruct((B,), jnp.float32),
    mesh=_mesh,
    scratch_types=[
        pltpu.VMEM((_NIDX,), jnp.int32),
        pltpu.VMEM((_NIDX,), jnp.float32),
        pltpu.VMEM((_BW,), jnp.float32),
        pltpu.SemaphoreType.DMA,
        pltpu.SemaphoreType.DMA,
    ],
)
def _sc_gather_sum(idx_hbm, table_hbm, out_hbm,
                   idx_v, vals_v, acc_v, sem, sem2):
    wid = lax.axis_index("s") * _NC + lax.axis_index("c")

    pltpu.sync_copy(idx_hbm.at[wid], idx_v)

    table_flat = table_hbm.at[0]
    _HALF = _NIDX // 2
    _FH = F // 2
    lo = pl.ds(0, _HALF)
    hi = pl.ds(_HALF, _HALF)
    pltpu.make_async_copy(table_flat.at[idx_v.at[lo]], vals_v.at[lo],
                          sem).start()
    pltpu.make_async_copy(table_flat.at[idx_v.at[hi]], vals_v.at[hi],
                          sem2).start()

    pltpu.make_async_copy(table_flat.at[idx_v.at[lo]], vals_v.at[lo],
                          sem).wait()

    def chunk_lo(c, carry):
        acc = vals_v[pl.ds(c * 16, 16)]
        for f in range(1, _FH):
            acc = acc + vals_v[pl.ds(f * _BW + c * 16, 16)]
        acc_v[pl.ds(c * 16, 16)] = acc
        return carry

    lax.fori_loop(0, _NCHUNK, chunk_lo, 0)

    pltpu.make_async_copy(table_flat.at[idx_v.at[hi]], vals_v.at[hi],
                          sem2).wait()

    def chunk_hi(c, carry):
        acc = acc_v[pl.ds(c * 16, 16)]
        for f in range(_FH, F):
            acc = acc + vals_v[pl.ds(f * _BW + c * 16, 16)]
        acc_v[pl.ds(c * 16, 16)] = acc
        return carry

    lax.fori_loop(0, _NCHUNK, chunk_hi, 0)

    pltpu.sync_copy(acc_v, out_hbm.at[pl.ds(wid * _BW, _BW)])


def _tc_combine_body(cat_ref, numt_ref, w_ref, b_ref, out_ref):
    acc = cat_ref[...] + b_ref[0]
    numt = numt_ref[...]
    for k in range(K):
        acc = acc + numt[k, :] * w_ref[0, k]
    out_ref[...] = acc


_BBLK = B


def _tc_combine(cat_sums, num_t, num_linear, bias):
    return pl.pallas_call(
        _tc_combine_body,
        out_shape=jax.ShapeDtypeStruct((B,), jnp.float32),
        grid=(B // _BBLK,),
        in_specs=[
            pl.BlockSpec((_BBLK,), lambda i: (i,)),
            pl.BlockSpec((K, _BBLK), lambda i: (0, i)),
            pl.BlockSpec(memory_space=pltpu.SMEM),
            pl.BlockSpec(memory_space=pltpu.SMEM),
        ],
        out_specs=pl.BlockSpec((_BBLK,), lambda i: (i,)),
    )(cat_sums, num_t, num_linear, bias)


def kernel(cat_features, num_features, cat_linear_weight, num_linear, bias):
    cat = cat_features.astype(jnp.int32)
    idx_r = (cat.reshape(_NW, _BW, F).transpose(0, 2, 1)
             .reshape(_NW, _NIDX))
    table2 = cat_linear_weight.astype(jnp.float32).reshape(1, V)
    cat_sums = _sc_gather_sum(idx_r, table2)
    num_t = num_features.astype(jnp.float32).T
    out = _tc_combine(cat_sums, num_t, num_linear.astype(jnp.float32),
                      bias.astype(jnp.float32))
    return out.reshape(B, 1)
